# Initial kernel scaffold; baseline (speedup 1.0000x reference)
import jax
import jax.numpy as jnp
from jax.experimental import pallas as pl


def kernel(x_rxdx, x_age_gender, table, W1, b1, W2, b2):
    raise NotImplementedError("write your pallas kernel here")


# speedup vs baseline: 2.5287x; 2.5287x over previous
import functools

import jax
import jax.numpy as jnp
from jax import lax
from jax.experimental import pallas as pl
from jax.experimental.pallas import tpu as pltpu
from jax.experimental.pallas import tpu_sc as plsc

B = 4096
HIST = 200
D = 128
FLAT = HIST * D
OUT = 512

NC = 2
NS = 16
NW = NC * NS
TOTAL_ROWS = B * HIST
ROWS_PER_W = TOTAL_ROWS // NW
CHUNK = 128
NCHUNK = ROWS_PER_W // CHUNK


def _sc_gather(idx_hbm, table_hbm, out_hbm, idx_v, rows_v, sem):
    wid = lax.axis_index("s") * NC + lax.axis_index("c")
    base = wid * ROWS_PER_W

    def body(i, carry):
        off = base + i * CHUNK
        pltpu.sync_copy(idx_hbm.at[pl.ds(off, CHUNK)], idx_v)
        pltpu.async_copy(table_hbm.at[idx_v], rows_v, sem).wait()
        pltpu.sync_copy(rows_v, out_hbm.at[pl.ds(off, CHUNK)])
        return carry

    lax.fori_loop(0, NCHUNK, body, 0)


_sc_gather_call = functools.partial(
    pl.kernel,
    out_type=jax.ShapeDtypeStruct((TOTAL_ROWS, D), jnp.float32),
    mesh=plsc.VectorSubcoreMesh(core_axis_name="c", subcore_axis_name="s"),
    scratch_types=[
        pltpu.VMEM((CHUNK,), jnp.int32),
        pltpu.VMEM((CHUNK, D), jnp.float32),
        pltpu.SemaphoreType.DMA,
    ],
)(_sc_gather)


BM = 256
BK = 3200
NKT = FLAT // BK
NMT = B // BM


def _tc_body(x_ref, w1_ref, b1_ref, xa_ref, w2_ref, b2_ref,
             o1_ref, o2_ref, acc_ref):
    k = pl.program_id(0)
    m = pl.program_id(1)
    part = jnp.dot(x_ref[...], w1_ref[...], preferred_element_type=jnp.float32)

    @pl.when(k == 0)
    def _():
        acc_ref[m] = part

    @pl.when(k > 0)
    def _():
        acc_ref[m] += part

    @pl.when(k == NKT - 1)
    def _():
        o1_ref[...] = acc_ref[m] + b1_ref[...]

    o2_ref[...] = (jnp.dot(xa_ref[...], w2_ref[...],
                           preferred_element_type=jnp.float32) + b2_ref[...])


def kernel(x_rxdx, x_age_gender, table, W1, b1, W2, b2):
    idx_flat = x_rxdx.reshape(TOTAL_ROWS)
    h2_rows = _sc_gather_call(idx_flat, table)
    h2 = h2_rows.reshape(B, FLAT)

    o1, o2 = pl.pallas_call(
        _tc_body,
        grid=(NKT, NMT),
        in_specs=[
            pl.BlockSpec((BM, BK), lambda k, m: (m, k)),
            pl.BlockSpec((BK, OUT), lambda k, m: (k, 0)),
            pl.BlockSpec((1, OUT), lambda k, m: (0, 0)),
            pl.BlockSpec((BM, 2), lambda k, m: (m, 0)),
            pl.BlockSpec((2, 2), lambda k, m: (0, 0)),
            pl.BlockSpec((1, 2), lambda k, m: (0, 0)),
        ],
        out_specs=[
            pl.BlockSpec((BM, OUT), lambda k, m: (m, 0)),
            pl.BlockSpec((BM, 2), lambda k, m: (m, 0)),
        ],
        out_shape=[
            jax.ShapeDtypeStruct((B, OUT), jnp.float32),
            jax.ShapeDtypeStruct((B, 2), jnp.float32),
        ],
        scratch_shapes=[pltpu.VMEM((NMT, BM, OUT), jnp.float32)],
        compiler_params=pltpu.CompilerParams(
            dimension_semantics=("arbitrary", "arbitrary")),
    )(h2, W1, b1.reshape(1, OUT), x_age_gender, W2, b2.reshape(1, 2))

    return jnp.concatenate([o1, o2], axis=1)


# speedup vs baseline: 3.8508x; 1.5229x over previous
import functools

import jax
import jax.numpy as jnp
from jax import lax
from jax.experimental import pallas as pl
from jax.experimental.pallas import tpu as pltpu
from jax.experimental.pallas import tpu_sc as plsc

B = 4096
HIST = 200
D = 128
FLAT = HIST * D
OUT = 512

NC = 2
NS = 16
NW = NC * NS
SAMPLES_PER_W = B // NW
SPLIT = 104


def _sc_gather(idx_hbm, table_hbm, out_hbm, idx_v, rows_v, sem):
    wid = lax.axis_index("s") * NC + lax.axis_index("c")
    base_b = wid * SAMPLES_PER_W

    def body(i, carry):
        b = base_b + i
        pltpu.sync_copy(idx_hbm.at[b], idx_v)
        cp1 = pltpu.async_copy(
            table_hbm.at[idx_v.at[pl.ds(0, SPLIT)]],
            rows_v.at[pl.ds(0, SPLIT)], sem)
        cp2 = pltpu.async_copy(
            table_hbm.at[idx_v.at[pl.ds(SPLIT, HIST - SPLIT)]],
            rows_v.at[pl.ds(SPLIT, HIST - SPLIT)], sem)
        cp1.wait()
        cp2.wait()
        pltpu.sync_copy(rows_v, out_hbm.at[b])
        return carry

    lax.fori_loop(0, SAMPLES_PER_W, body, 0)


_sc_gather_call = functools.partial(
    pl.kernel,
    out_type=jax.ShapeDtypeStruct((B, HIST, D), jnp.float32),
    mesh=plsc.VectorSubcoreMesh(core_axis_name="c", subcore_axis_name="s"),
    scratch_types=[
        pltpu.VMEM((HIST,), jnp.int32),
        pltpu.VMEM((HIST, D), jnp.float32),
        pltpu.SemaphoreType.DMA,
    ],
    compiler_params=pltpu.CompilerParams(use_tc_tiling_on_sc=True),
)(_sc_gather)


BM = 256
HK = 40
NKT = HIST // HK
NMT = B // BM


def _tc_body(x_ref, w1_ref, b1_ref, xa_ref, w2_ref, b2_ref,
             o1_ref, o2_ref, acc_ref):
    k = pl.program_id(0)
    m = pl.program_id(1)
    part = jnp.dot(x_ref[:, 0, :], w1_ref[pl.ds(0, D), :],
                   preferred_element_type=jnp.float32)
    for h in range(1, HK):
        part += jnp.dot(x_ref[:, h, :], w1_ref[pl.ds(h * D, D), :],
                        preferred_element_type=jnp.float32)

    @pl.when(k == 0)
    def _():
        acc_ref[m] = part

    @pl.when(k > 0)
    def _():
        acc_ref[m] += part

    @pl.when(k == NKT - 1)
    def _():
        o1_ref[...] = acc_ref[m] + b1_ref[...]

    o2_ref[...] = (jnp.dot(xa_ref[...], w2_ref[...],
                           preferred_element_type=jnp.float32) + b2_ref[...])


def kernel(x_rxdx, x_age_gender, table, W1, b1, W2, b2):
    h1 = _sc_gather_call(x_rxdx, table)

    o1, o2 = pl.pallas_call(
        _tc_body,
        grid=(NKT, NMT),
        in_specs=[
            pl.BlockSpec((BM, HK, D), lambda k, m: (m, k, 0)),
            pl.BlockSpec((HK * D, OUT), lambda k, m: (k, 0)),
            pl.BlockSpec((1, OUT), lambda k, m: (0, 0)),
            pl.BlockSpec((BM, 2), lambda k, m: (m, 0)),
            pl.BlockSpec((2, 2), lambda k, m: (0, 0)),
            pl.BlockSpec((1, 2), lambda k, m: (0, 0)),
        ],
        out_specs=[
            pl.BlockSpec((BM, OUT), lambda k, m: (m, 0)),
            pl.BlockSpec((BM, 2), lambda k, m: (m, 0)),
        ],
        out_shape=[
            jax.ShapeDtypeStruct((B, OUT), jnp.float32),
            jax.ShapeDtypeStruct((B, 2), jnp.float32),
        ],
        scratch_shapes=[pltpu.VMEM((NMT, BM, OUT), jnp.float32)],
        compiler_params=pltpu.CompilerParams(
            dimension_semantics=("arbitrary", "arbitrary")),
    )(h1, W1, b1.reshape(1, OUT), x_age_gender, W2, b2.reshape(1, 2))

    return jnp.concatenate([o1, o2], axis=1)


# speedup vs baseline: 4.4646x; 1.1594x over previous
import functools

import jax
import jax.numpy as jnp
from jax import lax
from jax.experimental import pallas as pl
from jax.experimental.pallas import tpu as pltpu
from jax.experimental.pallas import tpu_sc as plsc

B = 4096
HIST = 200
D = 128
FLAT = HIST * D
OUT = 512

NCHK = 4
BC = B // NCHK

NC = 2
NS = 16
NW = NC * NS
SAMPLES_PER_W = BC // NW
SPLIT = 104


def _sc_gather(idx_hbm, table_hbm, out_hbm, idx_v, rows_v, sem):
    wid = lax.axis_index("s") * NC + lax.axis_index("c")
    base_b = wid * SAMPLES_PER_W

    def body(i, carry):
        b = base_b + i
        pltpu.sync_copy(idx_hbm.at[b], idx_v)
        cp1 = pltpu.async_copy(
            table_hbm.at[idx_v.at[pl.ds(0, SPLIT)]],
            rows_v.at[pl.ds(0, SPLIT)], sem)
        cp2 = pltpu.async_copy(
            table_hbm.at[idx_v.at[pl.ds(SPLIT, HIST - SPLIT)]],
            rows_v.at[pl.ds(SPLIT, HIST - SPLIT)], sem)
        cp1.wait()
        cp2.wait()
        pltpu.sync_copy(rows_v, out_hbm.at[b])
        return carry

    lax.fori_loop(0, SAMPLES_PER_W, body, 0)


_sc_gather_call = functools.partial(
    pl.kernel,
    out_type=jax.ShapeDtypeStruct((BC, HIST, D), jnp.float32),
    mesh=plsc.VectorSubcoreMesh(core_axis_name="c", subcore_axis_name="s"),
    scratch_types=[
        pltpu.VMEM((HIST,), jnp.int32),
        pltpu.VMEM((HIST, D), jnp.float32),
        pltpu.SemaphoreType.DMA,
    ],
    compiler_params=pltpu.CompilerParams(use_tc_tiling_on_sc=True),
)(_sc_gather)


BM = 256
HK = 40
NKT = HIST // HK
NMT = BC // BM


def _tc_body(x_ref, w1_ref, b1_ref, xa_ref, w2_ref, b2_ref,
             o_ref, acc_ref):
    k = pl.program_id(0)
    m = pl.program_id(1)
    part = jnp.dot(x_ref[:, 0, :], w1_ref[pl.ds(0, D), :],
                   preferred_element_type=jnp.float32)
    for h in range(1, HK):
        part += jnp.dot(x_ref[:, h, :], w1_ref[pl.ds(h * D, D), :],
                        preferred_element_type=jnp.float32)

    @pl.when(k == 0)
    def _():
        acc_ref[m] = part

    @pl.when(k > 0)
    def _():
        acc_ref[m] += part

    @pl.when(k == NKT - 1)
    def _():
        o_ref[:, :OUT] = acc_ref[m] + b1_ref[...]
        o_ref[:, OUT:OUT + 2] = (
            jnp.dot(xa_ref[...], w2_ref[...],
                    preferred_element_type=jnp.float32) + b2_ref[...])


def _tc_matmul(h1, W1, b1, xa, W2, b2):
    return pl.pallas_call(
        _tc_body,
        grid=(NKT, NMT),
        in_specs=[
            pl.BlockSpec((BM, HK, D), lambda k, m: (m, k, 0)),
            pl.BlockSpec((HK * D, OUT), lambda k, m: (k, 0)),
            pl.BlockSpec((1, OUT), lambda k, m: (0, 0)),
            pl.BlockSpec((BM, 2), lambda k, m: (m, 0)),
            pl.BlockSpec((2, 2), lambda k, m: (0, 0)),
            pl.BlockSpec((1, 2), lambda k, m: (0, 0)),
        ],
        out_specs=pl.BlockSpec((BM, OUT + 2), lambda k, m: (m, 0)),
        out_shape=jax.ShapeDtypeStruct((BC, OUT + 2), jnp.float32),
        scratch_shapes=[pltpu.VMEM((NMT, BM, OUT), jnp.float32)],
        compiler_params=pltpu.CompilerParams(
            dimension_semantics=("arbitrary", "arbitrary")),
    )(h1, W1, b1, xa, W2, b2)


def kernel(x_rxdx, x_age_gender, table, W1, b1, W2, b2):
    b1r = b1.reshape(1, OUT)
    b2r = b2.reshape(1, 2)
    outs = []
    for c in range(NCHK):
        idx_c = lax.slice_in_dim(x_rxdx, c * BC, (c + 1) * BC, axis=0)
        xa_c = lax.slice_in_dim(x_age_gender, c * BC, (c + 1) * BC, axis=0)
        h1_c = _sc_gather_call(idx_c, table)
        outs.append(_tc_matmul(h1_c, W1, b1r, xa_c, W2, b2r))
    return jnp.concatenate(outs, axis=0)


# speedup vs baseline: 4.4860x; 1.0048x over previous
import functools

import jax
import jax.numpy as jnp
from jax import lax
from jax.experimental import pallas as pl
from jax.experimental.pallas import tpu as pltpu
from jax.experimental.pallas import tpu_sc as plsc

B = 4096
HIST = 200
D = 128
FLAT = HIST * D
OUT = 512

NCHK = 4
BC = B // NCHK

NC = 2
NS = 16
NW = NC * NS
SAMPLES_PER_W = BC // NW
SPLIT = 104


def _sc_gather(idx_hbm, table_hbm, out_hbm, idx_v, rows_v, sem):
    wid = lax.axis_index("s") * NC + lax.axis_index("c")
    base_b = wid * SAMPLES_PER_W

    def body(i, carry):
        b = base_b + i
        pltpu.sync_copy(idx_hbm.at[b], idx_v)
        cp1 = pltpu.async_copy(
            table_hbm.at[idx_v.at[pl.ds(0, SPLIT)]],
            rows_v.at[pl.ds(0, SPLIT)], sem)
        cp2 = pltpu.async_copy(
            table_hbm.at[idx_v.at[pl.ds(SPLIT, HIST - SPLIT)]],
            rows_v.at[pl.ds(SPLIT, HIST - SPLIT)], sem)
        cp1.wait()
        cp2.wait()
        pltpu.sync_copy(rows_v, out_hbm.at[b])
        return carry

    lax.fori_loop(0, SAMPLES_PER_W, body, 0)


_sc_gather_call = functools.partial(
    pl.kernel,
    out_type=jax.ShapeDtypeStruct((BC, HIST, D), jnp.float32),
    mesh=plsc.VectorSubcoreMesh(core_axis_name="c", subcore_axis_name="s"),
    scratch_types=[
        pltpu.VMEM((HIST,), jnp.int32),
        pltpu.VMEM((HIST, D), jnp.float32),
        pltpu.SemaphoreType.DMA,
    ],
    compiler_params=pltpu.CompilerParams(use_tc_tiling_on_sc=True),
)(_sc_gather)


BM = 256
HK = 40
NKT = HIST // HK
NMT = BC // BM


def _tc_body(x_ref, w1_ref, b1_ref, xa_ref, w2_ref, b2_ref,
             o_ref, acc_ref):
    k = pl.program_id(0)
    m = pl.program_id(1)
    part = None
    for j in range(HK // 2):
        xcat = jnp.concatenate(
            [x_ref[:, 2 * j, :], x_ref[:, 2 * j + 1, :]],
            axis=1).astype(jnp.bfloat16)
        d = jnp.dot(xcat, w1_ref[pl.ds(j * 2 * D, 2 * D), :],
                    preferred_element_type=jnp.float32)
        part = d if part is None else part + d

    @pl.when(k == 0)
    def _():
        acc_ref[m] = part

    @pl.when(k > 0)
    def _():
        acc_ref[m] += part

    @pl.when(k == NKT - 1)
    def _():
        o_ref[:, :OUT] = acc_ref[m] + b1_ref[...]
        o_ref[:, OUT:OUT + 2] = (
            jnp.dot(xa_ref[...], w2_ref[...],
                    preferred_element_type=jnp.float32) + b2_ref[...])


def _tc_matmul(h1, W1bf, b1, xa, W2, b2):
    return pl.pallas_call(
        _tc_body,
        grid=(NKT, NMT),
        in_specs=[
            pl.BlockSpec((BM, HK, D), lambda k, m: (m, k, 0)),
            pl.BlockSpec((HK * D, OUT), lambda k, m: (k, 0)),
            pl.BlockSpec((1, OUT), lambda k, m: (0, 0)),
            pl.BlockSpec((BM, 2), lambda k, m: (m, 0)),
            pl.BlockSpec((2, 2), lambda k, m: (0, 0)),
            pl.BlockSpec((1, 2), lambda k, m: (0, 0)),
        ],
        out_specs=pl.BlockSpec((BM, OUT + 2), lambda k, m: (m, 0)),
        out_shape=jax.ShapeDtypeStruct((BC, OUT + 2), jnp.float32),
        scratch_shapes=[pltpu.VMEM((NMT, BM, OUT), jnp.float32)],
        compiler_params=pltpu.CompilerParams(
            dimension_semantics=("arbitrary", "arbitrary")),
    )(h1, W1bf, b1, xa, W2, b2)


def kernel(x_rxdx, x_age_gender, table, W1, b1, W2, b2):
    W1bf = W1.astype(jnp.bfloat16)
    b1r = b1.reshape(1, OUT)
    b2r = b2.reshape(1, 2)
    out = jnp.zeros((B, OUT + 2), jnp.float32)
    for c in range(NCHK):
        idx_c = lax.slice_in_dim(x_rxdx, c * BC, (c + 1) * BC, axis=0)
        xa_c = lax.slice_in_dim(x_age_gender, c * BC, (c + 1) * BC, axis=0)
        h1_c = _sc_gather_call(idx_c, table)
        o_c = _tc_matmul(h1_c, W1bf, b1r, xa_c, W2, b2r)
        out = lax.dynamic_update_slice(out, o_c, (c * BC, 0))
    return out
